# baseline (speedup 1.0000x reference)
import jax
import jax.numpy as jnp
from jax import lax
from jax.experimental import pallas as pl
from jax.experimental.pallas import tpu as pltpu
from jax.experimental.pallas import tpu_sc as plsc

_BATCH = 4096
_NN = 16384
_SY = 8192
_NW = 32
_ROWS_PER_W = _BATCH // _NW


def _sc_body(act_hbm, comb_hbm, out_hbm, comb_v, row_v, orow_v):
    c = lax.axis_index("c")
    s = lax.axis_index("s")
    wid = s * 2 + c
    base = wid * _ROWS_PER_W

    pltpu.sync_copy(comb_hbm, comb_v)

    def row_fn(r, carry):
        pltpu.sync_copy(act_hbm.at[base + r], row_v)

        def j_fn(j, carry2):
            cv = comb_v[pl.ds(j * 16, 16)]
            il = cv & 0xFFFF
            ir = cv >> 16
            a = plsc.load_gather(row_v, [il])
            b = plsc.load_gather(row_v, [ir])
            orow_v[pl.ds(j * 16, 16)] = a * b
            return carry2

        lax.fori_loop(0, _SY // 16, j_fn, None)
        pltpu.sync_copy(orow_v, out_hbm.at[base + r])
        return carry

    lax.fori_loop(0, _ROWS_PER_W, row_fn, None)


def kernel(post_activations, left_indices, right_indices):
    li = left_indices.astype(jnp.int32)
    ri = right_indices.astype(jnp.int32)
    comb = li | (ri << 16)

    mesh = plsc.VectorSubcoreMesh(core_axis_name="c", subcore_axis_name="s")
    f = pl.kernel(
        _sc_body,
        out_type=jax.ShapeDtypeStruct((_BATCH, _SY), jnp.float32),
        mesh=mesh,
        scratch_types=[
            pltpu.VMEM((_SY,), jnp.int32),
            pltpu.VMEM((_NN,), jnp.float32),
            pltpu.VMEM((_SY,), jnp.float32),
        ],
        compiler_params=pltpu.CompilerParams(needs_layout_passes=False),
    )
    return f(post_activations, comb)
